# Initial kernel scaffold; baseline (speedup 1.0000x reference)
import jax
import jax.numpy as jnp
from jax.experimental import pallas as pl


def kernel(x, edge_index, W1, b1, W2, b2):
    raise NotImplementedError("write your pallas kernel here")


# speedup vs baseline: 12.9718x; 12.9718x over previous
import functools

import jax
import jax.numpy as jnp
from jax import lax
from jax.experimental import pallas as pl
from jax.experimental.pallas import tpu as pltpu
from jax.experimental.pallas import tpu_sc as plsc

N = 10000
E = 320000
F = 128
H1 = 256
NC = 2
NS = 16
NW = NC * NS
EPW = E // NW
C = 80
NCHUNK = EPW // C
RPT = 624
RPT_SIZES = [C] * 7 + [RPT - 7 * C]
TAIL_OFF = NS * RPT
TAIL_ROWS = N - TAIL_OFF

_MESH = plsc.VectorSubcoreMesh(core_axis_name="c", subcore_axis_name="s")


def _make_sc_scatter(W, with_gather):
    scratch = [pltpu.VMEM((C,), jnp.int32)]
    if with_gather:
        scratch.append(pltpu.VMEM((C,), jnp.int32))
    scratch += [
        pltpu.VMEM((C, W), jnp.float32),
        pltpu.VMEM_SHARED((N, W), jnp.float32),
        pltpu.SemaphoreType.DMA,
    ]

    def body(*refs):
        if with_gather:
            (src_hbm, dst_hbm, feat_hbm, part_hbm,
             didx_v, sidx_v, rows_v, acc_s, gsem) = refs
        else:
            dst_hbm, part_hbm, didx_v, rows_v, acc_s, gsem = refs
        c = lax.axis_index("c")
        s = lax.axis_index("s")
        wid = s * NC + c
        z16 = jnp.zeros((16,), jnp.float32)
        g = W // 16

        def zr(i, _):
            rows_v[i // g, pl.ds((i % g) * 16, 16)] = z16
            return 0

        lax.fori_loop(0, C * g, zr, 0)

        off0 = s * RPT
        roff = 0
        for size in RPT_SIZES:
            pltpu.sync_copy(rows_v.at[pl.ds(0, size)],
                            acc_s.at[pl.ds(off0 + roff, size)])
            roff += size

        @pl.when(s == NS - 1)
        def _():
            pltpu.sync_copy(rows_v.at[pl.ds(0, TAIL_ROWS)],
                            acc_s.at[pl.ds(TAIL_OFF, TAIL_ROWS)])

        if not with_gather:
            o16 = jnp.ones((16,), jnp.float32)

            def fr(i, _):
                rows_v[i // g, pl.ds((i % g) * 16, 16)] = o16
                return 0

            lax.fori_loop(0, C * g, fr, 0)

        plsc.subcore_barrier()

        def chunk(i, _):
            base = wid * EPW + i * C
            pltpu.sync_copy(dst_hbm.at[pl.ds(base, C)], didx_v)
            if with_gather:
                pltpu.sync_copy(src_hbm.at[pl.ds(base, C)], sidx_v)
                pltpu.async_copy(feat_hbm.at[sidx_v], rows_v, gsem).wait()
            pltpu.sync_copy(rows_v, acc_s.at[didx_v], add=True)
            return 0

        lax.fori_loop(0, NCHUNK, chunk, 0)
        plsc.subcore_barrier()

        out0 = c * N + s * RPT
        roff = 0
        for size in RPT_SIZES:
            pltpu.sync_copy(acc_s.at[pl.ds(off0 + roff, size)],
                            part_hbm.at[pl.ds(out0 + roff, size)])
            roff += size

        @pl.when(s == NS - 1)
        def _():
            pltpu.sync_copy(acc_s.at[pl.ds(TAIL_OFF, TAIL_ROWS)],
                            part_hbm.at[pl.ds(c * N + TAIL_OFF, TAIL_ROWS)])

    return pl.kernel(
        body,
        out_type=jax.ShapeDtypeStruct((2 * N, W), jnp.float32),
        mesh=_MESH,
        scratch_types=scratch,
    )


_deg_call = _make_sc_scatter(F, False)
_agg_call = _make_sc_scatter(F, True)


R = 1000
GRID = N // R
NBLK = N // R


def _tc1_body(d0, d1, x0, dinv, xs):
    deg = d0[:, 0:1] + d1[:, 0:1] + 1.0
    dv = lax.rsqrt(deg)
    dinv[...] = dv
    xs[...] = dv * x0[...]


_tc1 = pl.pallas_call(
    _tc1_body,
    grid=(GRID,),
    in_specs=[
        pl.BlockSpec((R, F), lambda i: (i, 0)),
        pl.BlockSpec((R, F), lambda i: (i + NBLK, 0)),
        pl.BlockSpec((R, F), lambda i: (i, 0)),
    ],
    out_specs=[
        pl.BlockSpec((R, 1), lambda i: (i, 0)),
        pl.BlockSpec((R, F), lambda i: (i, 0)),
    ],
    out_shape=[
        jax.ShapeDtypeStruct((N, 1), jnp.float32),
        jax.ShapeDtypeStruct((N, F), jnp.float32),
    ],
)


def _tc2_body(dinv, p0, p1, xs, w1, b1, w2, ys):
    dv = dinv[...]
    a1 = dv * (p0[...] + p1[...] + xs[...])
    h = jnp.maximum(jnp.dot(a1, w1[...], preferred_element_type=jnp.float32)
                    + b1[...], 0.0)
    y = jnp.dot(h, w2[...], preferred_element_type=jnp.float32)
    ys[...] = dv * y


_tc2 = pl.pallas_call(
    _tc2_body,
    grid=(GRID,),
    in_specs=[
        pl.BlockSpec((R, 1), lambda i: (i, 0)),
        pl.BlockSpec((R, F), lambda i: (i, 0)),
        pl.BlockSpec((R, F), lambda i: (i + NBLK, 0)),
        pl.BlockSpec((R, F), lambda i: (i, 0)),
        pl.BlockSpec((F, H1), lambda i: (0, 0)),
        pl.BlockSpec((1, H1), lambda i: (0, 0)),
        pl.BlockSpec((H1, F), lambda i: (0, 0)),
    ],
    out_specs=pl.BlockSpec((R, F), lambda i: (i, 0)),
    out_shape=jax.ShapeDtypeStruct((N, F), jnp.float32),
)


def _tc3_body(dinv, q0, q1, ys, b2, out):
    out[...] = dinv[...] * (q0[...] + q1[...] + ys[...]) + b2[...]


_tc3 = pl.pallas_call(
    _tc3_body,
    grid=(GRID,),
    in_specs=[
        pl.BlockSpec((R, 1), lambda i: (i, 0)),
        pl.BlockSpec((R, F), lambda i: (i, 0)),
        pl.BlockSpec((R, F), lambda i: (i + NBLK, 0)),
        pl.BlockSpec((R, F), lambda i: (i, 0)),
        pl.BlockSpec((1, F), lambda i: (0, 0)),
    ],
    out_specs=pl.BlockSpec((R, F), lambda i: (i, 0)),
    out_shape=jax.ShapeDtypeStruct((N, F), jnp.float32),
)


def kernel(x, edge_index, W1, b1, W2, b2):
    x0 = x[:, 588:]
    src = edge_index[0]
    dst = edge_index[1]
    degp = _deg_call(dst)
    dinv, xs = _tc1(degp, degp, x0)
    p = _agg_call(src, dst, xs)
    ys = _tc2(dinv, p, p, xs, W1, b1.reshape(1, H1), W2)
    q = _agg_call(src, dst, ys)
    return _tc3(dinv, q, q, ys, b2.reshape(1, F))


# speedup vs baseline: 16.1707x; 1.2466x over previous
import functools

import jax
import jax.numpy as jnp
from jax import lax
from jax.experimental import pallas as pl
from jax.experimental.pallas import tpu as pltpu
from jax.experimental.pallas import tpu_sc as plsc

N = 10000
E = 320000
F = 128
H1 = 256
NC = 2
NS = 16
NW = NC * NS
EPW = E // NW
C = 40
NCHUNK = EPW // C
RPT = 624
ZPIECES = [C] * (RPT // C) + ([RPT % C] if RPT % C else [])
TAIL_OFF = NS * RPT
TAIL_ROWS = N - TAIL_OFF

_MESH = plsc.VectorSubcoreMesh(core_axis_name="c", subcore_axis_name="s")


NBUF = 4
LAG = 2


def _zero_acc(rows_v, acc_s, s):
    off0 = s * RPT
    roff = 0
    for size in ZPIECES:
        pltpu.sync_copy(rows_v.at[pl.ds(0, size)],
                        acc_s.at[pl.ds(off0 + roff, size)])
        roff += size

    @pl.when(s == NS - 1)
    def _():
        pltpu.sync_copy(rows_v.at[pl.ds(0, TAIL_ROWS)],
                        acc_s.at[pl.ds(TAIL_OFF, TAIL_ROWS)])


def _write_out(acc_s, out_hbm, c, s):
    off0 = s * RPT
    out0 = c * N + s * RPT
    pltpu.sync_copy(acc_s.at[pl.ds(off0, RPT)], out_hbm.at[pl.ds(out0, RPT)])

    @pl.when(s == NS - 1)
    def _():
        pltpu.sync_copy(acc_s.at[pl.ds(TAIL_OFF, TAIL_ROWS)],
                        out_hbm.at[pl.ds(c * N + TAIL_OFF, TAIL_ROWS)])


def _fill_rows(rows_v, nrows, W, val16):
    g = W // 16

    def fr(i, _):
        rows_v[i // g, pl.ds((i % g) * 16, 16)] = val16
        return 0

    lax.fori_loop(0, nrows * g, fr, 0)


def _make_deg(W):
    def body(dst_hbm, degp_hbm, d0, d1, d2, d3, rows_v, acc_s,
             s0, s1, s2, s3):
        didx = (d0, d1, d2, d3)
        ssem = (s0, s1, s2, s3)
        c = lax.axis_index("c")
        s = lax.axis_index("s")
        wid = s * NC + c
        _fill_rows(rows_v, C, W, jnp.zeros((16,), jnp.float32))
        _zero_acc(rows_v, acc_s, s)
        _fill_rows(rows_v, C, W, jnp.ones((16,), jnp.float32))
        plsc.subcore_barrier()

        def group(g, _):
            for t in range(NBUF):
                j = g * NBUF + t

                @pl.when(jnp.logical_and(j >= NBUF, j < NCHUNK))
                def _():
                    pltpu.make_async_copy(
                        rows_v, acc_s.at[didx[t]], ssem[t]).wait()

                @pl.when(j < NCHUNK)
                def _():
                    pltpu.sync_copy(dst_hbm.at[pl.ds(wid * EPW + j * C, C)],
                                    didx[t])
                    pltpu.async_copy(rows_v, acc_s.at[didx[t]], ssem[t],
                                     add=True)

            return 0

        lax.fori_loop(0, (NCHUNK + NBUF - 1) // NBUF, group, 0)
        for b in range(NBUF):
            pltpu.make_async_copy(rows_v, acc_s.at[didx[b]], ssem[b]).wait()
        plsc.subcore_barrier()
        _write_out(acc_s, degp_hbm, c, s)

    return pl.kernel(
        body,
        out_type=jax.ShapeDtypeStruct((2 * N, W), jnp.float32),
        mesh=_MESH,
        scratch_types=(
            [pltpu.VMEM((C,), jnp.int32)] * NBUF
            + [pltpu.VMEM((C, W), jnp.float32),
               pltpu.VMEM_SHARED((N, W), jnp.float32)]
            + [pltpu.SemaphoreType.DMA] * NBUF
        ),
    )


def _agg_body(src_hbm, dst_hbm, feat_hbm, part_hbm,
              si0, si1, si2, si3, di0, di1, di2, di3,
              rows0, rows1, rows2, rows3, acc_s,
              g0, g1, g2, g3, s0, s1, s2, s3):
    sidx = (si0, si1, si2, si3)
    didx = (di0, di1, di2, di3)
    rows = (rows0, rows1, rows2, rows3)
    gsem = (g0, g1, g2, g3)
    ssem = (s0, s1, s2, s3)
    c = lax.axis_index("c")
    s = lax.axis_index("s")
    wid = s * NC + c
    _fill_rows(rows0, C, F, jnp.zeros((16,), jnp.float32))
    _zero_acc(rows0, acc_s, s)
    plsc.subcore_barrier()

    def group(g, _):
        for t in range(NBUF):
            j = g * NBUF + t
            b_s = (t - LAG) % NBUF

            @pl.when(jnp.logical_and(j >= NBUF, j < NCHUNK))
            def _():
                pltpu.make_async_copy(
                    rows[t], acc_s.at[didx[t]], ssem[t]).wait()

            @pl.when(j < NCHUNK)
            def _():
                base = wid * EPW + j * C
                pltpu.sync_copy(src_hbm.at[pl.ds(base, C)], sidx[t])
                pltpu.sync_copy(dst_hbm.at[pl.ds(base, C)], didx[t])
                pltpu.async_copy(feat_hbm.at[sidx[t]], rows[t], gsem[t])

            @pl.when(jnp.logical_and(j >= LAG, j < NCHUNK + LAG))
            def _():
                pltpu.make_async_copy(feat_hbm.at[sidx[b_s]], rows[b_s],
                                      gsem[b_s]).wait()
                pltpu.async_copy(rows[b_s], acc_s.at[didx[b_s]],
                                 ssem[b_s], add=True)

        return 0

    n_slots = NCHUNK + LAG
    n_groups = (n_slots + NBUF - 1) // NBUF
    lax.fori_loop(0, n_groups, group, 0)
    for b in range(NBUF):
        pltpu.make_async_copy(rows[b], acc_s.at[didx[b]], ssem[b]).wait()
    plsc.subcore_barrier()
    _write_out(acc_s, part_hbm, c, s)


_agg_call = pl.kernel(
    _agg_body,
    out_type=jax.ShapeDtypeStruct((2 * N, F), jnp.float32),
    mesh=_MESH,
    scratch_types=(
        [pltpu.VMEM((C,), jnp.int32)] * (2 * NBUF)
        + [pltpu.VMEM((C, F), jnp.float32)] * NBUF
        + [pltpu.VMEM_SHARED((N, F), jnp.float32)]
        + [pltpu.SemaphoreType.DMA] * (2 * NBUF)
    ),
)

_deg_call = _make_deg(F)


R = 1000
GRID = N // R
NBLK = N // R


def _tc1_body(d0, d1, x0, dinv, xs):
    deg = d0[:, 0:1] + d1[:, 0:1] + 1.0
    dv = lax.rsqrt(deg)
    dinv[...] = dv
    xs[...] = dv * x0[...]


_tc1 = pl.pallas_call(
    _tc1_body,
    grid=(GRID,),
    in_specs=[
        pl.BlockSpec((R, F), lambda i: (i, 0)),
        pl.BlockSpec((R, F), lambda i: (i + NBLK, 0)),
        pl.BlockSpec((R, F), lambda i: (i, 0)),
    ],
    out_specs=[
        pl.BlockSpec((R, 1), lambda i: (i, 0)),
        pl.BlockSpec((R, F), lambda i: (i, 0)),
    ],
    out_shape=[
        jax.ShapeDtypeStruct((N, 1), jnp.float32),
        jax.ShapeDtypeStruct((N, F), jnp.float32),
    ],
)


def _tc2_body(dinv, p0, p1, xs, w1, b1, w2, ys):
    dv = dinv[...]
    a1 = dv * (p0[...] + p1[...] + xs[...])
    h = jnp.maximum(jnp.dot(a1, w1[...], preferred_element_type=jnp.float32)
                    + b1[...], 0.0)
    y = jnp.dot(h, w2[...], preferred_element_type=jnp.float32)
    ys[...] = dv * y


_tc2 = pl.pallas_call(
    _tc2_body,
    grid=(GRID,),
    in_specs=[
        pl.BlockSpec((R, 1), lambda i: (i, 0)),
        pl.BlockSpec((R, F), lambda i: (i, 0)),
        pl.BlockSpec((R, F), lambda i: (i + NBLK, 0)),
        pl.BlockSpec((R, F), lambda i: (i, 0)),
        pl.BlockSpec((F, H1), lambda i: (0, 0)),
        pl.BlockSpec((1, H1), lambda i: (0, 0)),
        pl.BlockSpec((H1, F), lambda i: (0, 0)),
    ],
    out_specs=pl.BlockSpec((R, F), lambda i: (i, 0)),
    out_shape=jax.ShapeDtypeStruct((N, F), jnp.float32),
)


def _tc3_body(dinv, q0, q1, ys, b2, out):
    out[...] = dinv[...] * (q0[...] + q1[...] + ys[...]) + b2[...]


_tc3 = pl.pallas_call(
    _tc3_body,
    grid=(GRID,),
    in_specs=[
        pl.BlockSpec((R, 1), lambda i: (i, 0)),
        pl.BlockSpec((R, F), lambda i: (i, 0)),
        pl.BlockSpec((R, F), lambda i: (i + NBLK, 0)),
        pl.BlockSpec((R, F), lambda i: (i, 0)),
        pl.BlockSpec((1, F), lambda i: (0, 0)),
    ],
    out_specs=pl.BlockSpec((R, F), lambda i: (i, 0)),
    out_shape=jax.ShapeDtypeStruct((N, F), jnp.float32),
)


def kernel(x, edge_index, W1, b1, W2, b2):
    x0 = x[:, 588:]
    src = edge_index[0]
    dst = edge_index[1]
    degp = _deg_call(dst)
    dinv, xs = _tc1(degp, degp, x0)
    p = _agg_call(src, dst, xs)
    ys = _tc2(dinv, p, p, xs, W1, b1.reshape(1, H1), W2)
    q = _agg_call(src, dst, ys)
    return _tc3(dinv, q, q, ys, b2.reshape(1, F))


# speedup vs baseline: 29.9031x; 1.8492x over previous
import functools

import jax
import jax.numpy as jnp
from jax import lax
from jax.experimental import pallas as pl
from jax.experimental.pallas import tpu as pltpu
from jax.experimental.pallas import tpu_sc as plsc

N = 10000
E = 320000
F = 128
H1 = 256
NC = 2
NS = 16
NW = NC * NS
EPW = E // NW
C = 40
NCHUNK = EPW // C
RPT = 624
ZPIECES = [C] * (RPT // C) + ([RPT % C] if RPT % C else [])
TAIL_OFF = NS * RPT
TAIL_ROWS = N - TAIL_OFF

_MESH = plsc.VectorSubcoreMesh(core_axis_name="c", subcore_axis_name="s")


NBUF = 4
LAG = 2
IBUF = 8
IPF = 4


def _zero_acc(rows_v, acc_s, s):
    off0 = s * RPT
    roff = 0
    for size in ZPIECES:
        pltpu.sync_copy(rows_v.at[pl.ds(0, size)],
                        acc_s.at[pl.ds(off0 + roff, size)])
        roff += size

    @pl.when(s == NS - 1)
    def _():
        pltpu.sync_copy(rows_v.at[pl.ds(0, TAIL_ROWS)],
                        acc_s.at[pl.ds(TAIL_OFF, TAIL_ROWS)])


def _write_out(acc_s, out_hbm, c, s):
    off0 = s * RPT
    out0 = c * N + s * RPT
    pltpu.sync_copy(acc_s.at[pl.ds(off0, RPT)], out_hbm.at[pl.ds(out0, RPT)])

    @pl.when(s == NS - 1)
    def _():
        pltpu.sync_copy(acc_s.at[pl.ds(TAIL_OFF, TAIL_ROWS)],
                        out_hbm.at[pl.ds(c * N + TAIL_OFF, TAIL_ROWS)])


def _fill_rows(rows_v, nrows, W, val16):
    g = W // 16

    def fr(i, _):
        rows_v[i // g, pl.ds((i % g) * 16, 16)] = val16
        return 0

    lax.fori_loop(0, nrows * g, fr, 0)


def _make_deg(W):
    def body(dst_hbm, degp_hbm, *refs):
        didx = refs[0:IBUF]
        rows_v = refs[IBUF]
        acc_s = refs[IBUF + 1]
        isem = refs[IBUF + 2:2 * IBUF + 2]
        ssem = refs[2 * IBUF + 2:]
        c = lax.axis_index("c")
        s = lax.axis_index("s")
        wid = s * NC + c
        _fill_rows(rows_v, C, W, jnp.zeros((16,), jnp.float32))
        _zero_acc(rows_v, acc_s, s)
        _fill_rows(rows_v, C, W, jnp.ones((16,), jnp.float32))
        plsc.subcore_barrier()

        def group(g, _):
            for t in range(IBUF):
                j = g * IBUF + t
                rb = t % NBUF
                ip = (t + IPF) % IBUF

                @pl.when(jnp.logical_and(j >= NBUF, j < NCHUNK))
                def _():
                    pltpu.make_async_copy(
                        rows_v, acc_s.at[didx[t]], ssem[rb]).wait()

                @pl.when(j + IPF < NCHUNK)
                def _():
                    pltpu.async_copy(
                        dst_hbm.at[pl.ds(wid * EPW + (j + IPF) * C, C)],
                        didx[ip], isem[ip])

                @pl.when(j < IPF)
                def _():
                    pltpu.sync_copy(dst_hbm.at[pl.ds(wid * EPW + j * C, C)],
                                    didx[t])

                @pl.when(jnp.logical_and(j >= IPF, j < NCHUNK))
                def _():
                    pltpu.make_async_copy(
                        dst_hbm.at[pl.ds(0, C)], didx[t], isem[t]).wait()

                @pl.when(j < NCHUNK)
                def _():
                    pltpu.async_copy(rows_v, acc_s.at[didx[t]], ssem[rb],
                                     add=True)

            return 0

        lax.fori_loop(0, (NCHUNK + IBUF - 1) // IBUF, group, 0)
        for b in range(NBUF):
            pltpu.make_async_copy(rows_v, acc_s.at[didx[b]], ssem[b]).wait()
        plsc.subcore_barrier()
        _write_out(acc_s, degp_hbm, c, s)

    return pl.kernel(
        body,
        out_type=jax.ShapeDtypeStruct((2 * N, W), jnp.float32),
        mesh=_MESH,
        scratch_types=(
            [pltpu.VMEM((C,), jnp.int32)] * IBUF
            + [pltpu.VMEM((C, W), jnp.float32),
               pltpu.VMEM_SHARED((N, W), jnp.float32)]
            + [pltpu.SemaphoreType.DMA] * (IBUF + NBUF)
        ),
    )


def _agg_body(src_hbm, dst_hbm, feat_hbm, part_hbm, *refs):
    sidx = refs[0:IBUF]
    didx = refs[IBUF:2 * IBUF]
    rows = refs[2 * IBUF:2 * IBUF + NBUF]
    acc_s = refs[2 * IBUF + NBUF]
    isem = refs[2 * IBUF + NBUF + 1:3 * IBUF + NBUF + 1]
    gsem = refs[3 * IBUF + NBUF + 1:3 * IBUF + 2 * NBUF + 1]
    ssem = refs[3 * IBUF + 2 * NBUF + 1:]
    c = lax.axis_index("c")
    s = lax.axis_index("s")
    wid = s * NC + c
    _fill_rows(rows[0], C, F, jnp.zeros((16,), jnp.float32))
    _zero_acc(rows[0], acc_s, s)
    plsc.subcore_barrier()

    def group(g, _):
        for t in range(IBUF):
            j = g * IBUF + t
            rb = t % NBUF
            ip = (t + IPF) % IBUF
            ts = (t - LAG) % IBUF
            rs = ts % NBUF

            @pl.when(jnp.logical_and(j >= NBUF, j < NCHUNK))
            def _():
                pltpu.make_async_copy(
                    rows[rb], acc_s.at[didx[t]], ssem[rb]).wait()

            @pl.when(j + IPF < NCHUNK)
            def _():
                base = wid * EPW + (j + IPF) * C
                pltpu.async_copy(src_hbm.at[pl.ds(base, C)], sidx[ip],
                                 isem[ip])
                pltpu.async_copy(dst_hbm.at[pl.ds(base, C)], didx[ip],
                                 isem[ip])

            @pl.when(j < IPF)
            def _():
                base = wid * EPW + j * C
                pltpu.sync_copy(src_hbm.at[pl.ds(base, C)], sidx[t])
                pltpu.sync_copy(dst_hbm.at[pl.ds(base, C)], didx[t])

            @pl.when(jnp.logical_and(j >= IPF, j < NCHUNK))
            def _():
                pltpu.make_async_copy(
                    src_hbm.at[pl.ds(0, C)], sidx[t], isem[t]).wait()
                pltpu.make_async_copy(
                    dst_hbm.at[pl.ds(0, C)], didx[t], isem[t]).wait()

            @pl.when(j < NCHUNK)
            def _():
                pltpu.async_copy(feat_hbm.at[sidx[t]], rows[rb], gsem[rb])

            @pl.when(jnp.logical_and(j >= LAG, j < NCHUNK + LAG))
            def _():
                pltpu.make_async_copy(feat_hbm.at[sidx[ts]], rows[rs],
                                      gsem[rs]).wait()
                pltpu.async_copy(rows[rs], acc_s.at[didx[ts]],
                                 ssem[rs], add=True)

        return 0

    n_slots = NCHUNK + LAG
    n_groups = (n_slots + IBUF - 1) // IBUF
    lax.fori_loop(0, n_groups, group, 0)
    for b in range(NBUF):
        pltpu.make_async_copy(rows[b], acc_s.at[didx[b]], ssem[b]).wait()
    plsc.subcore_barrier()
    _write_out(acc_s, part_hbm, c, s)


_agg_call = pl.kernel(
    _agg_body,
    out_type=jax.ShapeDtypeStruct((2 * N, F), jnp.float32),
    mesh=_MESH,
    scratch_types=(
        [pltpu.VMEM((C,), jnp.int32)] * (2 * IBUF)
        + [pltpu.VMEM((C, F), jnp.float32)] * NBUF
        + [pltpu.VMEM_SHARED((N, F), jnp.float32)]
        + [pltpu.SemaphoreType.DMA] * (IBUF + 2 * NBUF)
    ),
)

_deg_call = _make_deg(F)


R = 1000
GRID = N // R
NBLK = N // R


def _tc1_body(d0, d1, x0, dinv, xs):
    deg = d0[:, 0:1] + d1[:, 0:1] + 1.0
    dv = lax.rsqrt(deg)
    dinv[...] = dv
    xs[...] = dv * x0[...]


_tc1 = pl.pallas_call(
    _tc1_body,
    grid=(GRID,),
    in_specs=[
        pl.BlockSpec((R, F), lambda i: (i, 0)),
        pl.BlockSpec((R, F), lambda i: (i + NBLK, 0)),
        pl.BlockSpec((R, F), lambda i: (i, 0)),
    ],
    out_specs=[
        pl.BlockSpec((R, 1), lambda i: (i, 0)),
        pl.BlockSpec((R, F), lambda i: (i, 0)),
    ],
    out_shape=[
        jax.ShapeDtypeStruct((N, 1), jnp.float32),
        jax.ShapeDtypeStruct((N, F), jnp.float32),
    ],
)


def _tc2_body(dinv, p0, p1, xs, w1, b1, w2, ys):
    dv = dinv[...]
    a1 = dv * (p0[...] + p1[...] + xs[...])
    h = jnp.maximum(jnp.dot(a1, w1[...], preferred_element_type=jnp.float32)
                    + b1[...], 0.0)
    y = jnp.dot(h, w2[...], preferred_element_type=jnp.float32)
    ys[...] = dv * y


_tc2 = pl.pallas_call(
    _tc2_body,
    grid=(GRID,),
    in_specs=[
        pl.BlockSpec((R, 1), lambda i: (i, 0)),
        pl.BlockSpec((R, F), lambda i: (i, 0)),
        pl.BlockSpec((R, F), lambda i: (i + NBLK, 0)),
        pl.BlockSpec((R, F), lambda i: (i, 0)),
        pl.BlockSpec((F, H1), lambda i: (0, 0)),
        pl.BlockSpec((1, H1), lambda i: (0, 0)),
        pl.BlockSpec((H1, F), lambda i: (0, 0)),
    ],
    out_specs=pl.BlockSpec((R, F), lambda i: (i, 0)),
    out_shape=jax.ShapeDtypeStruct((N, F), jnp.float32),
)


def _tc3_body(dinv, q0, q1, ys, b2, out):
    out[...] = dinv[...] * (q0[...] + q1[...] + ys[...]) + b2[...]


_tc3 = pl.pallas_call(
    _tc3_body,
    grid=(GRID,),
    in_specs=[
        pl.BlockSpec((R, 1), lambda i: (i, 0)),
        pl.BlockSpec((R, F), lambda i: (i, 0)),
        pl.BlockSpec((R, F), lambda i: (i + NBLK, 0)),
        pl.BlockSpec((R, F), lambda i: (i, 0)),
        pl.BlockSpec((1, F), lambda i: (0, 0)),
    ],
    out_specs=pl.BlockSpec((R, F), lambda i: (i, 0)),
    out_shape=jax.ShapeDtypeStruct((N, F), jnp.float32),
)


def kernel(x, edge_index, W1, b1, W2, b2):
    x0 = x[:, 588:]
    src = edge_index[0]
    dst = edge_index[1]
    degp = _deg_call(dst)
    dinv, xs = _tc1(degp, degp, x0)
    p = _agg_call(src, dst, xs)
    ys = _tc2(dinv, p, p, xs, W1, b1.reshape(1, H1), W2)
    q = _agg_call(src, dst, ys)
    return _tc3(dinv, q, q, ys, b2.reshape(1, F))
